# baseline (speedup 1.0000x reference)
import jax
import jax.numpy as jnp
from jax import lax
from jax.experimental import pallas as pl
from jax.experimental.pallas import tpu as pltpu
from jax.experimental.pallas import tpu_sc as plsc

_N = 10000
_NP = 10240
_E = 320000
_D = 128
_H = 64
_STEPS = 4
_NS = 16
_CH = 128
_KB = 20
_NBLK = 8
_NCH = _KB * _NBLK
_EPT = _CH * _NCH
_EPAD = _NS * _EPT
_NPT = _NP // _NS
_RW = 64
_NQ = _NPT // _RW
_NBUF = 4

_BCAST_DNUMS = lax.GatherDimensionNumbers(
    offset_dims=(), collapsed_slice_dims=(0,), start_index_map=(0,))


def _body(xs, xbf, src2, dst2, wv2, alph, out, stbf,
          acc, sblk, dblk, wblk, rows, srow, abuf, sbuf, bbuf, avec,
          gsem, ssem, bsem):
    cid = lax.axis_index("c")
    sid = lax.axis_index("s")
    row_base = (cid * _NP).astype(jnp.int32)

    pltpu.sync_copy(alph, avec)
    a = avec[...]
    alpha_v = 1.0 / (1.0 + jnp.exp(-a))
    one_m = 1.0 - alpha_v

    zero = jnp.zeros((16,), jnp.float32)

    for q in range(_NQ):
        r0 = row_base + sid * _NPT + q * _RW
        pltpu.sync_copy(xs.at[pl.ds(r0, _RW)], sbuf)
        pltpu.sync_copy(sbuf, out.at[pl.ds(r0, _RW)])
        pltpu.sync_copy(xbf.at[pl.ds(r0, _RW)], bbuf)
        pltpu.sync_copy(bbuf, stbf.at[pl.ds(r0, _RW)])
    plsc.subcore_barrier()

    def _load_block(par, gb):
        pltpu.async_copy(src2.at[cid, gb], sblk[par], bsem[par])
        pltpu.async_copy(dst2.at[gb], dblk[par], bsem[par])
        pltpu.async_copy(wv2.at[gb], wblk[par], bsem[par])

    def _wait_block(par):
        pltpu.make_async_copy(src2.at[cid, 0], sblk[par], bsem[par]).wait()
        pltpu.make_async_copy(dst2.at[0], dblk[par], bsem[par]).wait()
        pltpu.make_async_copy(wv2.at[0], wblk[par], bsem[par]).wait()

    def _scale(b, wref, j):
        @pl.loop(0, _CH // 16)
        def _grp(g):
            w16 = wref[j, pl.ds(g * 16, 16)]
            for l in range(16):
                lidx = jnp.full((16, 1), l, dtype=jnp.int32)
                wb = lax.gather(
                    w16, lidx, _BCAST_DNUMS, (1,),
                    mode=lax.GatherScatterMode.PROMISE_IN_BOUNDS)
                e = g * 16 + l
                for k in range(_H // 32):
                    v = rows[b][e, pl.ds(32 * k, 32)]
                    a_, b_ = plsc.unpack(
                        v, format=plsc.PackFormat.INTERLEAVED,
                        preferred_element_type=jnp.float32)
                    srow[b][e, pl.ds(32 * k, 16)] = a_ * wb
                    srow[b][e, pl.ds(32 * k + 16, 16)] = b_ * wb

    def _drain_scatter(b):
        pltpu.make_async_copy(srow[b], acc.at[dblk[0].at[0]], ssem[b]).wait()

    def _wait_gather(b):
        pltpu.make_async_copy(
            stbf.at[sblk[0].at[0]], rows[b], gsem[b]).wait()

    @pl.loop(0, _STEPS)
    def _step(_):
        _load_block(0, sid * _NBLK)

        @pl.loop(0, _RW)
        def _zero_abuf(r):
            for j in range(_H // 16):
                abuf[r, pl.ds(j * 16, 16)] = zero

        for q in range(_NQ):
            pltpu.sync_copy(abuf, acc.at[pl.ds(sid * _NPT + q * _RW, _RW)])
        plsc.subcore_barrier()

        @pl.loop(0, _NBLK // 2)
        def _bpair(bp):
            for par in range(2):
                blk = 2 * bp + par
                gb = sid * _NBLK + blk
                _wait_block(par)

                @pl.when(blk + 1 < _NBLK)
                def _pfb():
                    _load_block(1 - par, gb + 1)

                pltpu.async_copy(stbf.at[sblk[par].at[0]], rows[0], gsem[0])
                pltpu.async_copy(stbf.at[sblk[par].at[1]], rows[1], gsem[1])

                @pl.loop(0, _KB // _NBUF)
                def _quad(p):
                    for b in range(_NBUF):
                        j = _NBUF * p + b
                        b2 = (b + 2) % _NBUF

                        @pl.when(j >= 2)
                        def _dr():
                            _drain_scatter(b2)

                        @pl.when(j + 2 < _KB)
                        def _pf():
                            pltpu.async_copy(
                                stbf.at[sblk[par].at[j + 2]], rows[b2],
                                gsem[b2])

                        _wait_gather(b)
                        _scale(b, wblk[par], j)
                        pltpu.async_copy(
                            srow[b], acc.at[dblk[par].at[j]], ssem[b],
                            add=True)

                _drain_scatter((_KB - 2) % _NBUF)
                _drain_scatter((_KB - 1) % _NBUF)

        plsc.subcore_barrier()

        for q in range(_NQ):
            r0 = sid * _NPT + q * _RW
            pltpu.sync_copy(acc.at[pl.ds(r0, _RW)], abuf)
            pltpu.sync_copy(out.at[pl.ds(row_base + r0, _RW)], sbuf)

            @pl.loop(0, _RW)
            def _combine(r):
                for g in range(_H // 32):
                    s0 = pl.ds(32 * g, 16)
                    s1 = pl.ds(32 * g + 16, 16)
                    a_ = one_m * sbuf[r, s0] + alpha_v * abuf[r, s0]
                    b_ = one_m * sbuf[r, s1] + alpha_v * abuf[r, s1]
                    sbuf[r, s0] = a_
                    sbuf[r, s1] = b_
                    bbuf[r, pl.ds(32 * g, 32)] = plsc.pack(
                        a_, b_, format=plsc.PackFormat.INTERLEAVED)

            pltpu.sync_copy(sbuf, out.at[pl.ds(row_base + r0, _RW)])
            pltpu.sync_copy(bbuf, stbf.at[pl.ds(row_base + r0, _RW)])
        plsc.subcore_barrier()


_mesh = plsc.VectorSubcoreMesh(core_axis_name="c", subcore_axis_name="s")

_sc_call = pl.kernel(
    _body,
    out_type=(
        jax.ShapeDtypeStruct((2 * _NP, _H), jnp.float32),
        jax.ShapeDtypeStruct((2 * _NP, _H), jnp.bfloat16),
    ),
    mesh=_mesh,
    compiler_params=pltpu.CompilerParams(
        use_tc_tiling_on_sc=False, needs_layout_passes=False),
    scratch_types=[
        pltpu.VMEM_SHARED((_NP, _H), jnp.float32),
        [pltpu.VMEM((_KB, _CH), jnp.int32)] * 2,
        [pltpu.VMEM((_KB, _CH), jnp.int32)] * 2,
        [pltpu.VMEM((_KB, _CH), jnp.float32)] * 2,
        [pltpu.VMEM((_CH, _H), jnp.bfloat16)] * _NBUF,
        [pltpu.VMEM((_CH, _H), jnp.float32)] * _NBUF,
        pltpu.VMEM((_RW, _H), jnp.float32),
        pltpu.VMEM((_RW, _H), jnp.float32),
        pltpu.VMEM((_RW, _H), jnp.bfloat16),
        pltpu.VMEM((16,), jnp.float32),
        [pltpu.SemaphoreType.DMA] * _NBUF,
        [pltpu.SemaphoreType.DMA] * _NBUF,
        [pltpu.SemaphoreType.DMA] * 2,
    ],
)


@jax.jit
def kernel(x, edge_index, edge_weight, alpha_train):
    xs = jnp.zeros((2 * _NP, _H), jnp.float32)
    xs = xs.at[:_N].set(x[:, :_H]).at[_NP:_NP + _N].set(x[:, _H:])
    t = xs.reshape(2 * _NP, _H // 32, 2, 16)
    xbf = t.transpose(0, 1, 3, 2).reshape(2 * _NP, _H).astype(jnp.bfloat16)
    pad = _EPAD - _E
    spread = (jnp.arange(pad, dtype=jnp.int32) * 37) % _N
    srcp = jnp.concatenate([edge_index[0], spread])
    dstp = jnp.concatenate([edge_index[1], spread])
    wp = jnp.concatenate([edge_weight, jnp.zeros((pad,), jnp.float32)])
    src2 = jnp.stack([srcp, srcp + _NP]).reshape(2, _NS * _NBLK, _KB, _CH)
    dst2 = dstp.reshape(_NS * _NBLK, _KB, _CH)
    w2 = wp.reshape(_NS * _NBLK, _KB, _CH)
    alph = jnp.full((16,), alpha_train, dtype=jnp.float32)
    outs, _ = _sc_call(xs, xbf, src2, dst2, w2, alph)
    return jnp.concatenate([outs[:_N], outs[_NP:_NP + _N]], axis=1)


# speedup vs baseline: 1.9516x; 1.9516x over previous
import jax
import jax.numpy as jnp
from jax import lax
from jax.experimental import pallas as pl
from jax.experimental.pallas import tpu as pltpu
from jax.experimental.pallas import tpu_sc as plsc

_N = 10000
_NP = 10240
_E = 320000
_D = 128
_H = 64
_STEPS = 4
_NS = 16
_CH = 128
_KB = 20
_NBLK = 8
_NCH = _KB * _NBLK
_EPT = _CH * _NCH
_EPAD = _NS * _EPT
_NPT = _NP // _NS
_RW = 128
_NQ = _NPT // _RW
_NBUF = 5

_BCAST_DNUMS = lax.GatherDimensionNumbers(
    offset_dims=(), collapsed_slice_dims=(0,), start_index_map=(0,))


def _body(xs, src2, dst2, wv2, alph, out,
          acc, sblk, dblk, wblk, rows, abuf, sbuf, avec, gsem, ssem, bsem):
    cid = lax.axis_index("c")
    sid = lax.axis_index("s")
    row_base = (cid * _NP).astype(jnp.int32)

    pltpu.sync_copy(alph, avec)
    a = avec[...]
    alpha_v = 1.0 / (1.0 + jnp.exp(-a))
    one_m = 1.0 - alpha_v

    zero = jnp.zeros((16,), jnp.float32)

    for q in range(_NQ):
        r0 = row_base + sid * _NPT + q * _RW
        pltpu.sync_copy(xs.at[pl.ds(r0, _RW)], sbuf)
        pltpu.sync_copy(sbuf, out.at[pl.ds(r0, _RW)])
    plsc.subcore_barrier()

    def _load_block(par, gb):
        pltpu.async_copy(src2.at[cid, gb], sblk[par], bsem[par])
        pltpu.async_copy(dst2.at[gb], dblk[par], bsem[par])
        pltpu.async_copy(wv2.at[gb], wblk[par], bsem[par])

    def _wait_block(par):
        pltpu.make_async_copy(src2.at[cid, 0], sblk[par], bsem[par]).wait()
        pltpu.make_async_copy(dst2.at[0], dblk[par], bsem[par]).wait()
        pltpu.make_async_copy(wv2.at[0], wblk[par], bsem[par]).wait()

    def _scale(rbuf, wref, j):
        @pl.loop(0, _CH // 16)
        def _grp(g):
            w16 = wref[j, pl.ds(g * 16, 16)]
            for l in range(16):
                lidx = jnp.full((16, 1), l, dtype=jnp.int32)
                wb = lax.gather(
                    w16, lidx, _BCAST_DNUMS, (1,),
                    mode=lax.GatherScatterMode.PROMISE_IN_BOUNDS)
                e = g * 16 + l
                for k in range(_H // 16):
                    sl = pl.ds(k * 16, 16)
                    rbuf[e, sl] = rbuf[e, sl] * wb

    def _drain_scatter(b):
        pltpu.make_async_copy(rows[b], acc.at[dblk[0].at[0]], ssem[b]).wait()

    def _wait_gather(b):
        pltpu.make_async_copy(
            out.at[sblk[0].at[0]], rows[b], gsem[b]).wait()

    @pl.loop(0, _STEPS)
    def _step(_):
        _load_block(0, sid * _NBLK)

        @pl.loop(0, _RW)
        def _zero_abuf(r):
            for j in range(_H // 16):
                abuf[r, pl.ds(j * 16, 16)] = zero

        for q in range(_NQ):
            pltpu.sync_copy(abuf, acc.at[pl.ds(sid * _NPT + q * _RW, _RW)])
        plsc.subcore_barrier()

        @pl.loop(0, _NBLK // 2)
        def _bpair(bp):
            for par in range(2):
                blk = 2 * bp + par
                gb = sid * _NBLK + blk
                _wait_block(par)

                @pl.when(blk + 1 < _NBLK)
                def _pfb():
                    _load_block(1 - par, gb + 1)

                pltpu.async_copy(out.at[sblk[par].at[0]], rows[0], gsem[0])
                pltpu.async_copy(out.at[sblk[par].at[1]], rows[1], gsem[1])
                pltpu.async_copy(out.at[sblk[par].at[2]], rows[2], gsem[2])

                @pl.loop(0, _KB // _NBUF)
                def _quint(p):
                    for b in range(_NBUF):
                        j = _NBUF * p + b
                        b2 = (b + 3) % _NBUF

                        @pl.when(j >= 2)
                        def _dr():
                            _drain_scatter(b2)

                        @pl.when(j + 3 < _KB)
                        def _pf():
                            pltpu.async_copy(
                                out.at[sblk[par].at[j + 3]], rows[b2],
                                gsem[b2])

                        _wait_gather(b)
                        _scale(rows[b], wblk[par], j)
                        pltpu.async_copy(
                            rows[b], acc.at[dblk[par].at[j]], ssem[b],
                            add=True)

                _drain_scatter((_KB - 2) % _NBUF)
                _drain_scatter((_KB - 1) % _NBUF)

        plsc.subcore_barrier()

        for q in range(_NQ):
            r0 = sid * _NPT + q * _RW
            pltpu.sync_copy(acc.at[pl.ds(r0, _RW)], abuf)
            pltpu.sync_copy(out.at[pl.ds(row_base + r0, _RW)], sbuf)

            @pl.loop(0, _RW)
            def _combine(r):
                for j in range(_H // 16):
                    sl = pl.ds(j * 16, 16)
                    sbuf[r, sl] = one_m * sbuf[r, sl] + alpha_v * abuf[r, sl]

            pltpu.sync_copy(sbuf, out.at[pl.ds(row_base + r0, _RW)])
        plsc.subcore_barrier()


_mesh = plsc.VectorSubcoreMesh(core_axis_name="c", subcore_axis_name="s")

_sc_call = pl.kernel(
    _body,
    out_type=jax.ShapeDtypeStruct((2 * _NP, _H), jnp.float32),
    mesh=_mesh,
    compiler_params=pltpu.CompilerParams(use_tc_tiling_on_sc=False),
    scratch_types=[
        pltpu.VMEM_SHARED((_NP, _H), jnp.float32),
        [pltpu.VMEM((_KB, _CH), jnp.int32)] * 2,
        [pltpu.VMEM((_KB, _CH), jnp.int32)] * 2,
        [pltpu.VMEM((_KB, _CH), jnp.float32)] * 2,
        [pltpu.VMEM((_CH, _H), jnp.float32)] * _NBUF,
        pltpu.VMEM((_RW, _H), jnp.float32),
        pltpu.VMEM((_RW, _H), jnp.float32),
        pltpu.VMEM((16,), jnp.float32),
        [pltpu.SemaphoreType.DMA] * _NBUF,
        [pltpu.SemaphoreType.DMA] * _NBUF,
        [pltpu.SemaphoreType.DMA] * 2,
    ],
)


@jax.jit
def kernel(x, edge_index, edge_weight, alpha_train):
    xs = jnp.zeros((2 * _NP, _H), jnp.float32)
    xs = xs.at[:_N].set(x[:, :_H]).at[_NP:_NP + _N].set(x[:, _H:])
    pad = _EPAD - _E
    spread = (jnp.arange(pad, dtype=jnp.int32) * 37) % _N
    srcp = jnp.concatenate([edge_index[0], spread])
    dstp = jnp.concatenate([edge_index[1], spread])
    wp = jnp.concatenate([edge_weight, jnp.zeros((pad,), jnp.float32)])
    src2 = jnp.stack([srcp, srcp + _NP]).reshape(2, _NS * _NBLK, _KB, _CH)
    dst2 = dstp.reshape(_NS * _NBLK, _KB, _CH)
    w2 = wp.reshape(_NS * _NBLK, _KB, _CH)
    alph = jnp.full((16,), alpha_train, dtype=jnp.float32)
    outs = _sc_call(xs, src2, dst2, w2, alph)
    return jnp.concatenate([outs[:_N], outs[_NP:_NP + _N]], axis=1)
